# Initial kernel scaffold; baseline (speedup 1.0000x reference)
import functools

import jax
import jax.numpy as jnp
from jax import lax
from jax.experimental import pallas as pl
from jax.experimental.pallas import tpu as pltpu
from jax.experimental.pallas import tpu_sc as plsc

NUM_PERTS = 100000
LATENT_DIM = 256
BATCH = 16384

_INFO = plsc.get_sparse_core_info()
_NC, _NS = _INFO.num_cores, _INFO.num_subcores
_NW = _NC * _NS
_BPW = BATCH // _NW
_CH = 128
_NCH = _BPW // _CH


def _sc_gather(pert_idx, table):
    mesh = plsc.VectorSubcoreMesh(core_axis_name="c", subcore_axis_name="s")

    @functools.partial(
        pl.kernel,
        mesh=mesh,
        out_type=jax.ShapeDtypeStruct((BATCH, LATENT_DIM), jnp.float32),
        scratch_types=[
            pltpu.VMEM((_BPW,), jnp.int32),
            pltpu.VMEM((_CH, LATENT_DIM), jnp.float32),
            pltpu.VMEM((_CH, LATENT_DIM), jnp.float32),
            pltpu.SemaphoreType.DMA,
            pltpu.SemaphoreType.DMA,
        ],
    )
    def gather_k(idx_hbm, table_hbm, out_hbm, idx_v, buf0, buf1, sem0, sem1):
        wid = lax.axis_index("s") * _NC + lax.axis_index("c")
        base = wid * _BPW
        pltpu.sync_copy(idx_hbm.at[pl.ds(base, _BPW)], idx_v)
        bufs = (buf0, buf1)
        sems = (sem0, sem1)
        cps = [None, None]
        cps[0] = pltpu.async_copy(
            table_hbm.at[idx_v.at[pl.ds(0, _CH)]], buf0, sem0)
        for c in range(_NCH):
            if c + 1 < _NCH:
                cps[(c + 1) % 2] = pltpu.async_copy(
                    table_hbm.at[idx_v.at[pl.ds((c + 1) * _CH, _CH)]],
                    bufs[(c + 1) % 2], sems[(c + 1) % 2])
            cps[c % 2].wait()
            pltpu.sync_copy(bufs[c % 2], out_hbm.at[pl.ds(base + c * _CH, _CH)])

    return gather_k(pert_idx, table)


def _tc_body(x_ref, w_ref, b_ref, g_ref, bt_ref, o_ref):
    x = x_ref[...]
    h = jnp.dot(x, w_ref[...], preferred_element_type=jnp.float32)
    h = h + b_ref[...]
    mean = jnp.mean(h, axis=1, keepdims=True)
    cen = h - mean
    var = jnp.mean(cen * cen, axis=1, keepdims=True)
    xhat = cen * lax.rsqrt(var + 1e-5)
    h2 = xhat * g_ref[...] + bt_ref[...]
    o_ref[...] = 0.5 * h2 * (1.0 + lax.erf(h2 * 0.7071067811865476))


def _tc_mlp(emb, W, b, gamma, beta, bm=2048):
    grid = (BATCH // bm,)
    return pl.pallas_call(
        _tc_body,
        grid=grid,
        in_specs=[
            pl.BlockSpec((bm, LATENT_DIM), lambda i: (i, 0)),
            pl.BlockSpec((LATENT_DIM, LATENT_DIM), lambda i: (0, 0)),
            pl.BlockSpec((1, LATENT_DIM), lambda i: (0, 0)),
            pl.BlockSpec((1, LATENT_DIM), lambda i: (0, 0)),
            pl.BlockSpec((1, LATENT_DIM), lambda i: (0, 0)),
        ],
        out_specs=pl.BlockSpec((bm, LATENT_DIM), lambda i: (i, 0)),
        out_shape=jax.ShapeDtypeStruct((BATCH, LATENT_DIM), jnp.float32),
    )(emb, W, b.reshape(1, LATENT_DIM), gamma.reshape(1, LATENT_DIM),
      beta.reshape(1, LATENT_DIM))


def kernel(pert_idx, table, W, b, gamma, beta):
    idx = pert_idx.astype(jnp.int32)
    emb = _sc_gather(idx, table)
    return _tc_mlp(emb, W, b, gamma, beta)


# speedup vs baseline: 7.6268x; 7.6268x over previous
import functools

import jax
import jax.numpy as jnp
from jax import lax
from jax.experimental import pallas as pl
from jax.experimental.pallas import tpu as pltpu
from jax.experimental.pallas import tpu_sc as plsc

NUM_PERTS = 100000
LATENT_DIM = 256
BATCH = 16384

_NC, _NS = 2, 16
_NW = _NC * _NS
_BPW = BATCH // _NW
_CH = 128
_NCH = _BPW // _CH


def _sc_gather(pert_idx, table):
    mesh = plsc.VectorSubcoreMesh(core_axis_name="c", subcore_axis_name="s")

    @functools.partial(
        pl.kernel,
        mesh=mesh,
        out_type=jax.ShapeDtypeStruct((BATCH, LATENT_DIM), jnp.float32),
        scratch_types=[
            pltpu.VMEM((_BPW,), jnp.int32),
            pltpu.VMEM((_CH, LATENT_DIM), jnp.float32),
            pltpu.VMEM((_CH, LATENT_DIM), jnp.float32),
            pltpu.SemaphoreType.DMA,
            pltpu.SemaphoreType.DMA,
        ],
    )
    def gather_k(idx_hbm, table_hbm, out_hbm, idx_v, buf0, buf1, sem0, sem1):
        wid = lax.axis_index("s") * _NC + lax.axis_index("c")
        base = wid * _BPW
        pltpu.sync_copy(idx_hbm.at[pl.ds(base, _BPW)], idx_v)
        bufs = (buf0, buf1)
        sems = (sem0, sem1)
        cps = [None, None]
        cps[0] = pltpu.async_copy(
            table_hbm.at[idx_v.at[pl.ds(0, _CH)]], buf0, sem0)
        for c in range(_NCH):
            if c + 1 < _NCH:
                cps[(c + 1) % 2] = pltpu.async_copy(
                    table_hbm.at[idx_v.at[pl.ds((c + 1) * _CH, _CH)]],
                    bufs[(c + 1) % 2], sems[(c + 1) % 2])
            cps[c % 2].wait()
            pltpu.sync_copy(bufs[c % 2], out_hbm.at[pl.ds(base + c * _CH, _CH)])

    return gather_k(pert_idx, table)


def _tc_body(x_ref, w_ref, b_ref, g_ref, bt_ref, o_ref):
    x = x_ref[...]
    h = jnp.dot(x, w_ref[...], preferred_element_type=jnp.float32)
    h = h + b_ref[...]
    mean = jnp.mean(h, axis=1, keepdims=True)
    cen = h - mean
    var = jnp.mean(cen * cen, axis=1, keepdims=True)
    xhat = cen * lax.rsqrt(var + 1e-5)
    h2 = xhat * g_ref[...] + bt_ref[...]
    o_ref[...] = 0.5 * h2 * (1.0 + lax.erf(h2 * 0.7071067811865476))


def _tc_mlp(emb, W, b, gamma, beta, bm=2048):
    grid = (BATCH // bm,)
    return pl.pallas_call(
        _tc_body,
        grid=grid,
        in_specs=[
            pl.BlockSpec((bm, LATENT_DIM), lambda i: (i, 0)),
            pl.BlockSpec((LATENT_DIM, LATENT_DIM), lambda i: (0, 0)),
            pl.BlockSpec((1, LATENT_DIM), lambda i: (0, 0)),
            pl.BlockSpec((1, LATENT_DIM), lambda i: (0, 0)),
            pl.BlockSpec((1, LATENT_DIM), lambda i: (0, 0)),
        ],
        out_specs=pl.BlockSpec((bm, LATENT_DIM), lambda i: (i, 0)),
        out_shape=jax.ShapeDtypeStruct((BATCH, LATENT_DIM), jnp.float32),
    )(emb, W, b.reshape(1, LATENT_DIM), gamma.reshape(1, LATENT_DIM),
      beta.reshape(1, LATENT_DIM))


def kernel(pert_idx, table, W, b, gamma, beta):
    idx = pert_idx.astype(jnp.int32)
    emb = _sc_gather(idx, table)
    return _tc_mlp(emb, W, b, gamma, beta)


# speedup vs baseline: 7.7254x; 1.0129x over previous
import functools

import jax
import jax.numpy as jnp
from jax import lax
from jax.experimental import pallas as pl
from jax.experimental.pallas import tpu as pltpu
from jax.experimental.pallas import tpu_sc as plsc

NUM_PERTS = 100000
LATENT_DIM = 256
BATCH = 16384

_NC, _NS = 2, 16
_NW = _NC * _NS
_BPW = BATCH // _NW
_CH = 128
_NCH = _BPW // _CH


def _sc_gather(pert_idx, table, nrows):
    mesh = plsc.VectorSubcoreMesh(core_axis_name="c", subcore_axis_name="s")
    bpw = nrows // _NW
    nch = max(1, bpw // _CH)
    ch = min(bpw, _CH)

    @functools.partial(
        pl.kernel,
        mesh=mesh,
        out_type=jax.ShapeDtypeStruct((nrows, LATENT_DIM), jnp.float32),
        scratch_types=[
            pltpu.VMEM((bpw,), jnp.int32),
            pltpu.VMEM((ch, LATENT_DIM), jnp.float32),
            pltpu.VMEM((ch, LATENT_DIM), jnp.float32),
            pltpu.SemaphoreType.DMA,
            pltpu.SemaphoreType.DMA,
        ],
    )
    def gather_k(idx_hbm, table_hbm, out_hbm, idx_v, buf0, buf1, sem0, sem1):
        wid = lax.axis_index("s") * _NC + lax.axis_index("c")
        base = wid * bpw
        pltpu.sync_copy(idx_hbm.at[pl.ds(base, bpw)], idx_v)
        bufs = (buf0, buf1)
        sems = (sem0, sem1)
        cps = [None, None]
        cps[0] = pltpu.async_copy(
            table_hbm.at[idx_v.at[pl.ds(0, ch)]], buf0, sem0)
        for c in range(nch):
            if c + 1 < nch:
                cps[(c + 1) % 2] = pltpu.async_copy(
                    table_hbm.at[idx_v.at[pl.ds((c + 1) * ch, ch)]],
                    bufs[(c + 1) % 2], sems[(c + 1) % 2])
            cps[c % 2].wait()
            pltpu.sync_copy(bufs[c % 2], out_hbm.at[pl.ds(base + c * ch, ch)])

    return gather_k(pert_idx, table)


def _tc_body(x_ref, w_ref, b_ref, g_ref, bt_ref, o_ref):
    x = x_ref[...]
    h = jnp.dot(x, w_ref[...], preferred_element_type=jnp.float32)
    h = h + b_ref[...]
    mean = jnp.mean(h, axis=1, keepdims=True)
    cen = h - mean
    var = jnp.mean(cen * cen, axis=1, keepdims=True)
    xhat = cen * lax.rsqrt(var + 1e-5)
    h2 = xhat * g_ref[...] + bt_ref[...]
    o_ref[...] = 0.5 * h2 * (1.0 + lax.erf(h2 * 0.7071067811865476))


def _tc_body_carry(x_ref, w_ref, b_ref, g_ref, bt_ref, prev_ref, o_ref):
    del prev_ref
    _tc_body(x_ref, w_ref, b_ref, g_ref, bt_ref, o_ref)


def _tc_mlp_chunk(emb, W, b, gamma, beta, prev, row_off, bm):
    nb = emb.shape[0] // bm
    off_b = row_off // bm
    in_specs = [
        pl.BlockSpec((bm, LATENT_DIM), lambda i: (i, 0)),
        pl.BlockSpec((LATENT_DIM, LATENT_DIM), lambda i: (0, 0)),
        pl.BlockSpec((1, LATENT_DIM), lambda i: (0, 0)),
        pl.BlockSpec((1, LATENT_DIM), lambda i: (0, 0)),
        pl.BlockSpec((1, LATENT_DIM), lambda i: (0, 0)),
    ]
    args = [emb, W, b.reshape(1, LATENT_DIM), gamma.reshape(1, LATENT_DIM),
            beta.reshape(1, LATENT_DIM)]
    kwargs = {}
    body = _tc_body
    if prev is not None:
        in_specs.append(pl.BlockSpec(memory_space=pl.ANY))
        args.append(prev)
        kwargs["input_output_aliases"] = {5: 0}
        body = _tc_body_carry
    return pl.pallas_call(
        body,
        grid=(nb,),
        in_specs=in_specs,
        out_specs=pl.BlockSpec((bm, LATENT_DIM),
                               lambda i, _o=off_b: (i + _o, 0)),
        out_shape=jax.ShapeDtypeStruct((BATCH, LATENT_DIM), jnp.float32),
        **kwargs,
    )(*args)


_NCHUNK = 2
_CB = BATCH // _NCHUNK
_BM = 2048


def kernel(pert_idx, table, W, b, gamma, beta):
    idx = pert_idx.astype(jnp.int32)
    embs = [_sc_gather(lax.slice(idx, (c * _CB,), ((c + 1) * _CB,)), table,
                       _CB) for c in range(_NCHUNK)]
    out = None
    for c in range(_NCHUNK):
        out = _tc_mlp_chunk(embs[c], W, b, gamma, beta, out, c * _CB, _BM)
    return out
